# Initial kernel scaffold; baseline (speedup 1.0000x reference)
import jax
import jax.numpy as jnp
from jax.experimental import pallas as pl


def kernel(features, img_feat, edge_index, params):
    raise NotImplementedError("write your pallas kernel here")


# speedup vs baseline: 4.8031x; 4.8031x over previous
import dataclasses
import functools
import jax
import jax.numpy as jnp
from jax import lax
from jax.experimental import pallas as pl
from jax.experimental.pallas import tpu as pltpu
from jax.experimental.pallas import tpu_sc as plsc

N = 10000
D = 128
E = 320000

NP = 10240
EP = 327680
NWORK = 32
EW = EP // NWORK
CB = 128
NSLICE = NP // 16
SCALE = 1.0 / (D ** 0.5)

_mesh = plsc.VectorSubcoreMesh(core_axis_name="c", subcore_axis_name="s")

_sc_params = pltpu.CompilerParams()
if "needs_layout_passes" in pltpu.CompilerParams.__dataclass_fields__:
    _sc_params = dataclasses.replace(_sc_params, needs_layout_passes=False)


_GD = lax.GatherDimensionNumbers(
    offset_dims=(), collapsed_slice_dims=(0,), start_index_map=(0,))


def _shuf(x, idx):
    return lax.gather(x, idx[:, None], _GD, slice_sizes=(1,),
                      mode=lax.GatherScatterMode.PROMISE_IN_BOUNDS)


def _seg_scan(sk, sv, lane, op):
    for s in (1, 2, 4, 8):
        idxs = jnp.maximum(lane - s, 0)
        kk = _shuf(sk, idxs)
        vv = _shuf(sv, idxs)
        ok = (lane >= s) & (kk == sk)
        sv = jnp.where(ok, op(sv, vv), sv)
    nk = _shuf(sk, jnp.minimum(lane + 1, 15))
    is_last = (sk != nk) | (lane == 15)
    return sv, is_last


def _alpha_kernel(q, k, dst, src):
    @functools.partial(
        pl.kernel,
        out_type=(
            jax.ShapeDtypeStruct((EP,), jnp.float32),
            jax.ShapeDtypeStruct((2, NP), jnp.float32),
        ),
        mesh=_mesh,
        compiler_params=_sc_params,
        scratch_types=[
            pltpu.VMEM((CB,), jnp.int32),
            pltpu.VMEM((CB,), jnp.int32),
            pltpu.VMEM((CB, D), jnp.float32),
            pltpu.VMEM((CB, D), jnp.float32),
            pltpu.VMEM((CB,), jnp.float32),
            pltpu.VMEM((NP,), jnp.float32),
            pltpu.VMEM((16, NSLICE), jnp.float32),
            pltpu.VMEM_SHARED((16, NP), jnp.float32),
        ],
    )
    def kern(q_hbm, k_hbm, dst_hbm, src_hbm, alpha_hbm, amax_hbm,
             dstb, srcb, qbuf, kbuf, alphab, amax_l, redbuf, stage):
        cid = lax.axis_index("c")
        sid = lax.axis_index("s")
        wid = cid * 16 + sid
        ebase = wid * EW
        lane = lax.iota(jnp.int32, 16)
        neg = jnp.full((16,), -1e30, jnp.float32)

        @pl.loop(0, NP, step=16)
        def _(i):
            amax_l[pl.ds(i, 16)] = neg

        @pl.loop(0, EW, step=CB)
        def _(c):
            pltpu.sync_copy(dst_hbm.at[pl.ds(ebase + c, CB)], dstb)
            pltpu.sync_copy(src_hbm.at[pl.ds(ebase + c, CB)], srcb)
            pltpu.sync_copy(q_hbm.at[dstb], qbuf)
            pltpu.sync_copy(k_hbm.at[srcb], kbuf)

            @pl.loop(0, CB, step=16)
            def _(g):
                def dot16(i, alphas):
                    e = g + i
                    acc = qbuf[e, pl.ds(0, 16)] * kbuf[e, pl.ds(0, 16)]
                    for j in range(1, 8):
                        acc = acc + qbuf[e, pl.ds(j * 16, 16)] * kbuf[e, pl.ds(j * 16, 16)]
                    tot = jnp.sum(acc) * SCALE
                    return jnp.where(lane == i, tot, alphas)

                av = lax.fori_loop(0, 16, dot16, jnp.zeros((16,), jnp.float32))
                alphab[pl.ds(g, 16)] = av
                dstv = dstb[pl.ds(g, 16)]
                sk, sv = plsc.sort_key_val(dstv, av)
                sv, is_last = _seg_scan(sk, sv, lane, jnp.maximum)
                cur = plsc.load_gather(amax_l, [sk], mask=is_last)
                plsc.store_scatter(amax_l, [sk], jnp.maximum(cur, sv), mask=is_last)

            pltpu.sync_copy(alphab, alpha_hbm.at[pl.ds(ebase + c, CB)])

        pltpu.sync_copy(amax_l, stage.at[sid])
        plsc.subcore_barrier()
        nb = sid * NSLICE
        for t in range(16):
            pltpu.sync_copy(stage.at[t, pl.ds(nb, NSLICE)], redbuf.at[t])

        @pl.loop(0, NSLICE, step=16)
        def _(i):
            m = redbuf[0, pl.ds(i, 16)]
            for t in range(1, 16):
                m = jnp.maximum(m, redbuf[t, pl.ds(i, 16)])
            redbuf[0, pl.ds(i, 16)] = m

        pltpu.sync_copy(redbuf.at[0], amax_hbm.at[cid, pl.ds(nb, NSLICE)])

    return kern(q, k, dst, src)


def _agg_kernel(v, dst, src, alpha, amax_part):
    @functools.partial(
        pl.kernel,
        out_type=(
            jax.ShapeDtypeStruct((2, NP, D), jnp.float32),
            jax.ShapeDtypeStruct((2, NP), jnp.float32),
        ),
        mesh=_mesh,
        compiler_params=_sc_params,
        scratch_types=[
            pltpu.VMEM((CB,), jnp.int32),
            pltpu.VMEM((CB,), jnp.int32),
            pltpu.VMEM((CB, D), jnp.float32),
            pltpu.VMEM((CB,), jnp.float32),
            pltpu.VMEM((CB,), jnp.float32),
            pltpu.VMEM((NP,), jnp.float32),
            pltpu.VMEM((NP,), jnp.float32),
            pltpu.VMEM((NP // CB, CB), jnp.int32),
            pltpu.VMEM_SHARED((NP,), jnp.float32),
            pltpu.VMEM_SHARED((NP, D), jnp.float32),
        ],
    )
    def kern(v_hbm, dst_hbm, src_hbm, alpha_hbm, amaxp_hbm, acc_hbm, den_hbm,
             dstb, srcb, vbuf, alphab, exb, amax_g, denom_l, idx2d, denom_sp, accum):
        cid = lax.axis_index("c")
        sid = lax.axis_index("s")
        wid = cid * 16 + sid
        ebase = wid * EW
        lane = lax.iota(jnp.int32, 16)
        zero = jnp.zeros((16,), jnp.float32)

        pltpu.sync_copy(amaxp_hbm.at[0], amax_g)
        pltpu.sync_copy(amaxp_hbm.at[1], denom_l)

        @pl.loop(0, NP, step=16)
        def _(i):
            amax_g[pl.ds(i, 16)] = jnp.maximum(amax_g[pl.ds(i, 16)],
                                               denom_l[pl.ds(i, 16)])
            denom_l[pl.ds(i, 16)] = zero

        @pl.loop(0, NP // CB)
        def _(c):
            for m in range(8):
                idx2d[c, pl.ds(m * 16, 16)] = lane + (c * CB + m * 16)

        @pl.when(sid == 0)
        def _():
            pltpu.sync_copy(denom_l, denom_sp)

        @pl.loop(0, CB)
        def _(r):
            for j in range(8):
                vbuf[r, pl.ds(j * 16, 16)] = zero

        nb = sid * NSLICE
        for b in range(NSLICE // CB):
            pltpu.sync_copy(vbuf, accum.at[pl.ds(nb + b * CB, CB)])
        plsc.subcore_barrier()

        @pl.loop(0, EW, step=CB)
        def _(c):
            pltpu.sync_copy(dst_hbm.at[pl.ds(ebase + c, CB)], dstb)
            pltpu.sync_copy(src_hbm.at[pl.ds(ebase + c, CB)], srcb)
            pltpu.sync_copy(alpha_hbm.at[pl.ds(ebase + c, CB)], alphab)
            pltpu.sync_copy(v_hbm.at[srcb], vbuf)

            @pl.loop(0, CB, step=16)
            def _(g):
                dstv = dstb[pl.ds(g, 16)]
                av = alphab[pl.ds(g, 16)]
                am = plsc.load_gather(amax_g, [dstv])
                ex = jnp.exp(av - am)
                exb[pl.ds(g, 16)] = ex
                sk, sv = plsc.sort_key_val(dstv, ex)
                sv, is_last = _seg_scan(sk, sv, lane, lambda a, b: a + b)
                plsc.addupdate_scatter(denom_l, [sk], sv, mask=is_last)

            @pl.loop(0, CB)
            def _(e):
                s16 = plsc.load_gather(exb, [jnp.full((16,), e, jnp.int32)])
                for j in range(8):
                    vbuf[e, pl.ds(j * 16, 16)] = vbuf[e, pl.ds(j * 16, 16)] * s16

            pltpu.sync_copy(vbuf, accum.at[dstb], add=True)

        @pl.loop(0, NP // CB)
        def _(c):
            pltpu.sync_copy(denom_l.at[pl.ds(c * CB, CB)],
                            denom_sp.at[idx2d.at[c]], add=True)

        plsc.subcore_barrier()
        pltpu.sync_copy(accum.at[pl.ds(nb, NSLICE)],
                        acc_hbm.at[cid, pl.ds(nb, NSLICE)])

        @pl.when(sid == 0)
        def _():
            pltpu.sync_copy(denom_sp, den_hbm.at[cid])

    return kern(v, dst, src, alpha, amax_part)


def _mm_body(x_ref, w_ref, b_ref, oq, ok_, ov, os_):
    res = jnp.dot(x_ref[...], w_ref[...], preferred_element_type=jnp.float32)
    res = res + b_ref[...]
    oq[...] = res[:, 0:D]
    ok_[...] = res[:, D:2 * D]
    ov[...] = res[:, 2 * D:3 * D]
    os_[...] = res[:, 3 * D:4 * D]


def _proj(x, wall, ball):
    blk = 1280
    grid = NP // blk
    out = jax.ShapeDtypeStruct((NP, D), jnp.float32)
    return pl.pallas_call(
        _mm_body,
        grid=(grid,),
        in_specs=[
            pl.BlockSpec((blk, D), lambda i: (i, 0)),
            pl.BlockSpec((D, 4 * D), lambda i: (0, 0)),
            pl.BlockSpec((1, 4 * D), lambda i: (0, 0)),
        ],
        out_specs=[pl.BlockSpec((blk, D), lambda i: (i, 0))] * 4,
        out_shape=[out] * 4,
    )(x, wall, ball)


def _comb_body(elu, acc_ref, den_ref, skip_ref, o_ref):
    a = acc_ref[0] + acc_ref[1]
    d = den_ref[0] + den_ref[1] + 1e-16
    out = a / d[:, None] + skip_ref[...]
    if elu:
        out = jnp.where(out > 0, out, jnp.exp(jnp.minimum(out, 0.0)) - 1.0)
    o_ref[...] = out


def _combine(acc, den, skip, elu):
    blk = 1280
    grid = NP // blk
    return pl.pallas_call(
        functools.partial(_comb_body, elu),
        grid=(grid,),
        in_specs=[
            pl.BlockSpec((2, blk, D), lambda i: (0, i, 0)),
            pl.BlockSpec((2, blk), lambda i: (0, i)),
            pl.BlockSpec((blk, D), lambda i: (i, 0)),
        ],
        out_specs=pl.BlockSpec((blk, D), lambda i: (i, 0)),
        out_shape=jax.ShapeDtypeStruct((NP, D), jnp.float32),
    )(acc, den, skip)


def _layer(x, wall, ball, dst, src, elu):
    q, k, v, s = _proj(x, wall, ball)
    alpha, amax_part = _alpha_kernel(q, k, dst, src)
    acc, den = _agg_kernel(v, dst, src, alpha, amax_part)
    return _combine(acc, den, s, elu)


def kernel(features, img_feat, edge_index, params):
    del features
    pad_e = EP - E
    dst = jnp.concatenate([
        edge_index[1],
        (jnp.arange(pad_e, dtype=jnp.int32) % (NP - N)) + N,
    ])
    src = jnp.concatenate([edge_index[0], jnp.zeros((pad_e,), jnp.int32)])
    x = jnp.pad(img_feat, ((0, NP - N), (0, 0)))

    walls, balls = [], []
    for (Wq, bq, Wk, bk, Wv, bv, Ws, bs) in params:
        walls.append(jnp.concatenate([Wq.T, Wk.T, Wv.T, Ws.T], axis=1))
        balls.append(jnp.concatenate([bq, bk, bv, bs]).reshape(1, 4 * D))

    x1 = _layer(x, walls[0], balls[0], dst, src, elu=True)
    x2 = _layer(x1, walls[1], balls[1], dst, src, elu=False)
    x3 = _layer(x2, walls[2], balls[2], dst, src, elu=True)
    x4 = _layer(x3, walls[3], balls[3], dst, src, elu=False)
    return (x2[:N], x4[:N])
